# baseline (speedup 1.0000x reference)
import functools

import jax
import jax.numpy as jnp
from jax import lax
from jax.experimental import pallas as pl
from jax.experimental.pallas import tpu as pltpu
from jax.experimental.pallas import tpu_sc as plsc

T = 2
N = 10000
E = 320000
D = 128

NC = 2
NS = 16
NW = NC * NS

NP = 10240
CHUNK = 128
EPT = 10112
NCH = EPT // CHUNK
E_PAD = NW * EPT

ZR = 8
IDX_BLK = 40
PHASES = ((0, 40), (40, 39))
ROWS_PER_TILE = NP // NS
DEG_SLICE = T * NP // NS

BLK = 1280
GRID = NP // BLK

_mesh = plsc.VectorSubcoreMesh(
    core_axis_name="c", subcore_axis_name="s", num_cores=NC, num_subcores=NS)


@functools.partial(
    pl.kernel,
    out_type=jax.ShapeDtypeStruct((NC, T * NP), jnp.float32),
    mesh=_mesh,
    scratch_types=[
        pltpu.VMEM((T * NCH, CHUNK), jnp.int32),
        pltpu.VMEM((CHUNK,), jnp.float32),
        pltpu.VMEM((DEG_SLICE,), jnp.float32),
        pltpu.VMEM_SHARED((T * NP,), jnp.float32),
        pltpu.SemaphoreType.DMA,
    ],
)
def _deg_kernel(dst_hbm, out_hbm, idx_v, ones_v, buf_v, deg_sh, sem):
    c = lax.axis_index("c")
    s = lax.axis_index("s")
    wid = s * NC + c
    for i in range(CHUNK // 16):
        ones_v[pl.ds(i * 16, 16)] = jnp.ones((16,), jnp.float32)
    for i in range(DEG_SLICE // 16):
        buf_v[pl.ds(i * 16, 16)] = jnp.zeros((16,), jnp.float32)
    pltpu.sync_copy(buf_v, deg_sh.at[pl.ds(s * DEG_SLICE, DEG_SLICE)])
    pltpu.sync_copy(dst_hbm.at[wid], idx_v)
    plsc.subcore_barrier()

    @pl.loop(0, T * NCH)
    def _scatter(j):
        pltpu.sync_copy(ones_v, deg_sh.at[idx_v.at[j]], add=True)

    plsc.subcore_barrier()
    pltpu.sync_copy(deg_sh.at[pl.ds(s * DEG_SLICE, DEG_SLICE)], buf_v)
    pltpu.sync_copy(buf_v, out_hbm.at[c, pl.ds(s * DEG_SLICE, DEG_SLICE)])


@functools.partial(
    pl.kernel,
    out_type=jax.ShapeDtypeStruct((NC, NP, D), jnp.float32),
    mesh=_mesh,
    scratch_types=[
        pltpu.VMEM((IDX_BLK, CHUNK), jnp.int32),
        pltpu.VMEM((IDX_BLK, CHUNK), jnp.int32),
        pltpu.VMEM((2, CHUNK, D), jnp.float32),
        pltpu.VMEM((ZR, D), jnp.float32),
        pltpu.VMEM_SHARED((NP, D), jnp.float32),
        pltpu.SemaphoreType.DMA,
        pltpu.SemaphoreType.DMA,
    ],
)
def _agg_kernel(hs_hbm, src_hbm, dst_hbm, out_hbm, srcv, dstv, rows, zbuf,
                acc, gsem, ssem):
    c = lax.axis_index("c")
    s = lax.axis_index("s")
    wid = s * NC + c
    for r in range(ZR):
        for k in range(D // 16):
            zbuf[r, pl.ds(k * 16, 16)] = jnp.zeros((16,), jnp.float32)
    base = s * ROWS_PER_TILE
    for i in range(ROWS_PER_TILE // ZR):
        pltpu.sync_copy(zbuf, acc.at[pl.ds(base + i * ZR, ZR)])
    plsc.subcore_barrier()

    def _drain(sem):
        pltpu.make_async_copy(hs_hbm.at[pl.ds(0, CHUNK)], rows.at[0],
                              sem).wait()

    for off, cnt in PHASES:
        pltpu.sync_copy(src_hbm.at[wid, pl.ds(off, cnt)],
                        srcv.at[pl.ds(0, cnt)])
        pltpu.sync_copy(dst_hbm.at[wid, pl.ds(off, cnt)],
                        dstv.at[pl.ds(0, cnt)])
        pltpu.async_copy(hs_hbm.at[srcv.at[0]], rows.at[0], gsem)
        pltpu.async_copy(hs_hbm.at[srcv.at[1]], rows.at[1], gsem)

        @pl.loop(0, cnt)
        def _edges(j):
            _drain(gsem)
            pltpu.async_copy(rows.at[j % 2], acc.at[dstv.at[j]], ssem,
                             add=True)
            @pl.when(j + 2 < cnt)
            def _():
                _drain(ssem)
                pltpu.async_copy(hs_hbm.at[srcv.at[j + 2]],
                                 rows.at[j % 2], gsem)

        _drain(ssem)
        _drain(ssem)

    plsc.subcore_barrier()
    for i in range(ROWS_PER_TILE // ZR):
        pltpu.sync_copy(acc.at[pl.ds(base + i * ZR, ZR)], zbuf)
        pltpu.sync_copy(zbuf, out_hbm.at[c].at[pl.ds(base + i * ZR, ZR)])


def _row_spec():
    return pl.BlockSpec((BLK, D), lambda i: (i, 0))


def _col_spec():
    return pl.BlockSpec((BLK, 1), lambda i: (i, 0))


def _full_spec(shape):
    return pl.BlockSpec(shape, lambda i: tuple(0 for _ in shape))


def _tc_scale_mm_body(x_ref, w_ref, p0_ref, p1_ref, o_ref):
    dinv = lax.rsqrt(p0_ref[...] + p1_ref[...] + 1.0)
    h = jnp.dot(x_ref[...], w_ref[...], preferred_element_type=jnp.float32)
    o_ref[...] = h * dinv


def _tc_scale_mm(x, w, p0, p1):
    return pl.pallas_call(
        _tc_scale_mm_body,
        grid=(GRID,),
        in_specs=[_row_spec(), _full_spec((D, D)), _col_spec(), _col_spec()],
        out_specs=_row_spec(),
        out_shape=jax.ShapeDtypeStruct((NP, D), jnp.float32),
    )(x, w, p0, p1)


def _tc_mid_body(q0_ref, q1_ref, hs_ref, p0_ref, p1_ref, b_ref, w_ref, o_ref):
    dinv = lax.rsqrt(p0_ref[...] + p1_ref[...] + 1.0)
    o1 = dinv * (q0_ref[...] + q1_ref[...] + hs_ref[...]) + b_ref[...]
    o1 = jnp.maximum(o1, 0.0)
    h = jnp.dot(o1, w_ref[...], preferred_element_type=jnp.float32)
    o_ref[...] = h * dinv


def _tc_mid(q0, q1, hs, p0, p1, b, w):
    return pl.pallas_call(
        _tc_mid_body,
        grid=(GRID,),
        in_specs=[_row_spec(), _row_spec(), _row_spec(), _col_spec(),
                  _col_spec(), _full_spec((1, D)), _full_spec((D, D))],
        out_specs=_row_spec(),
        out_shape=jax.ShapeDtypeStruct((NP, D), jnp.float32),
    )(q0, q1, hs, p0, p1, b, w)


def _tc_final_body(q0_ref, q1_ref, hs_ref, p0_ref, p1_ref, b_ref, wc_ref,
                   bc_ref, o_ref, imp_ref):
    dinv = lax.rsqrt(p0_ref[...] + p1_ref[...] + 1.0)
    o2 = dinv * (q0_ref[...] + q1_ref[...] + hs_ref[...]) + b_ref[...]
    o_ref[...] = o2
    imp_ref[...] = jnp.dot(o2, wc_ref[...],
                           preferred_element_type=jnp.float32) + bc_ref[...]


def _tc_final(q0, q1, hs, p0, p1, b, wc, bc):
    return pl.pallas_call(
        _tc_final_body,
        grid=(GRID,),
        in_specs=[_row_spec(), _row_spec(), _row_spec(), _col_spec(),
                  _col_spec(), _full_spec((1, D)), _full_spec((D, 1)),
                  _full_spec((1, 1))],
        out_specs=[_row_spec(), _col_spec()],
        out_shape=[jax.ShapeDtypeStruct((NP, D), jnp.float32),
                   jax.ShapeDtypeStruct((NP, 1), jnp.float32)],
    )(q0, q1, hs, p0, p1, b, wc, bc)


def kernel(x_seq, edge_indices, W1, b1, W2, b2, Wc, bc):
    x_pad = jnp.pad(x_seq, ((0, 0), (0, NP - N), (0, 0)))
    pad_idx = N + (jnp.arange(E_PAD - E, dtype=jnp.int32) % (NP - N))

    def _tiled(a):
        return jnp.concatenate([a, pad_idx]).reshape(NW, NCH, CHUNK)

    src_t = [_tiled(edge_indices[t, 0]) for t in range(T)]
    dst_t = [_tiled(edge_indices[t, 1]) for t in range(T)]
    deg_idx = jnp.concatenate([dst_t[0], dst_t[1] + NP], axis=1)

    degp = _deg_kernel(deg_idx)
    b1r = b1.reshape(1, D)
    b2r = b2.reshape(1, D)
    bcr = bc.reshape(1, 1)

    outs = []
    imp = None
    for t in range(T):
        p0 = degp[0, t * NP:(t + 1) * NP, None]
        p1 = degp[1, t * NP:(t + 1) * NP, None]
        hs1 = _tc_scale_mm(x_pad[t], W1, p0, p1)
        q = _agg_kernel(hs1, src_t[t], dst_t[t])
        hs2 = _tc_mid(q[0], q[1], hs1, p0, p1, b1r, W2)
        q2 = _agg_kernel(hs2, src_t[t], dst_t[t])
        o2, imp_t = _tc_final(q2[0], q2[1], hs2, p0, p1, b2r, Wc, bcr)
        outs.append(o2[:N])
        imp = imp_t
    return imp[:N, 0], jnp.stack(outs)


# speedup vs baseline: 1.0512x; 1.0512x over previous
import functools

import jax
import jax.numpy as jnp
from jax import lax
from jax.experimental import pallas as pl
from jax.experimental.pallas import tpu as pltpu
from jax.experimental.pallas import tpu_sc as plsc

T = 2
N = 10000
E = 320000
D = 128

NC = 2
NS = 16
NW = NC * NS

NP = 10240
TNP = T * NP
CHUNK = 128
EPT = 10112
NCH = EPT // CHUNK
E_PAD = NW * EPT

ZR = 32
IDX_BLK = 40
PHASES = ((0, 40), (40, 39))
ROWS_PER_TILE = NP // NS
WCH = ROWS_PER_TILE // CHUNK
DEG_SLICE = TNP // NS
DEG_WIN = 8

BLK = 1280
GRID = TNP // BLK

_mesh = plsc.VectorSubcoreMesh(
    core_axis_name="c", subcore_axis_name="s", num_cores=NC, num_subcores=NS)


@functools.partial(
    pl.kernel,
    out_type=jax.ShapeDtypeStruct((NC, TNP), jnp.float32),
    mesh=_mesh,
    scratch_types=[
        pltpu.VMEM((T * NCH, CHUNK), jnp.int32),
        pltpu.VMEM((CHUNK,), jnp.float32),
        pltpu.VMEM((DEG_SLICE,), jnp.float32),
        pltpu.VMEM_SHARED((TNP,), jnp.float32),
        pltpu.SemaphoreType.DMA,
    ],
)
def _deg_kernel(dst_hbm, out_hbm, idx_v, ones_v, buf_v, deg_sh, dsem):
    c = lax.axis_index("c")
    s = lax.axis_index("s")
    wid = s * NC + c
    for i in range(CHUNK // 16):
        ones_v[pl.ds(i * 16, 16)] = jnp.ones((16,), jnp.float32)
    for i in range(DEG_SLICE // 16):
        buf_v[pl.ds(i * 16, 16)] = jnp.zeros((16,), jnp.float32)
    pltpu.sync_copy(buf_v, deg_sh.at[pl.ds(s * DEG_SLICE, DEG_SLICE)])
    pltpu.sync_copy(dst_hbm.at[wid], idx_v)
    plsc.subcore_barrier()

    def _drain():
        pltpu.make_async_copy(out_hbm.at[0, pl.ds(0, CHUNK)], ones_v,
                              dsem).wait()

    @pl.loop(0, T * NCH)
    def _scatter(j):
        pltpu.async_copy(ones_v, deg_sh.at[idx_v.at[j]], dsem, add=True)

        @pl.when(j >= DEG_WIN)
        def _():
            _drain()

    for _ in range(DEG_WIN):
        _drain()
    plsc.subcore_barrier()
    pltpu.sync_copy(deg_sh.at[pl.ds(s * DEG_SLICE, DEG_SLICE)], buf_v)
    pltpu.sync_copy(buf_v, out_hbm.at[c, pl.ds(s * DEG_SLICE, DEG_SLICE)])


@functools.partial(
    pl.kernel,
    out_type=jax.ShapeDtypeStruct((NC, T, NP, D), jnp.float32),
    mesh=_mesh,
    scratch_types=[
        pltpu.VMEM((IDX_BLK, CHUNK), jnp.int32),
        pltpu.VMEM((IDX_BLK, CHUNK), jnp.int32),
        pltpu.VMEM((2, CHUNK, D), jnp.float32),
        pltpu.VMEM((ZR, D), jnp.float32),
        pltpu.VMEM_SHARED((NP, D), jnp.float32),
        pltpu.SemaphoreType.DMA,
        pltpu.SemaphoreType.DMA,
    ],
)
def _agg_kernel(hs_hbm, src_hbm, dst_hbm, out_hbm, srcv, dstv, rows, zbuf,
                acc, gsem, ssem):
    c = lax.axis_index("c")
    s = lax.axis_index("s")
    wid = s * NC + c
    base = s * ROWS_PER_TILE

    for r in range(ZR):
        for k in range(D // 16):
            zbuf[r, pl.ds(k * 16, 16)] = jnp.zeros((16,), jnp.float32)

    def _drain(sem):
        pltpu.make_async_copy(hs_hbm.at[pl.ds(0, CHUNK)], rows.at[0],
                              sem).wait()

    def _zero_acc():
        for i in range(ROWS_PER_TILE // ZR):
            pltpu.sync_copy(zbuf, acc.at[pl.ds(base + i * ZR, ZR)])

    _zero_acc()
    plsc.subcore_barrier()

    for t in range(T):
        for off, cnt in PHASES:
            pltpu.sync_copy(src_hbm.at[t, wid, pl.ds(off, cnt)],
                            srcv.at[pl.ds(0, cnt)])
            pltpu.sync_copy(dst_hbm.at[t, wid, pl.ds(off, cnt)],
                            dstv.at[pl.ds(0, cnt)])
            pltpu.async_copy(hs_hbm.at[srcv.at[0]], rows.at[0], gsem)
            pltpu.async_copy(hs_hbm.at[srcv.at[1]], rows.at[1], gsem)

            @pl.loop(0, cnt)
            def _edges(j):
                _drain(gsem)
                pltpu.async_copy(rows.at[j % 2], acc.at[dstv.at[j]], ssem,
                                 add=True)
                @pl.when(j + 2 < cnt)
                def _():
                    _drain(ssem)
                    pltpu.async_copy(hs_hbm.at[srcv.at[j + 2]],
                                     rows.at[j % 2], gsem)

            _drain(ssem)
            _drain(ssem)

        plsc.subcore_barrier()
        for i in range(WCH):
            pltpu.sync_copy(acc.at[pl.ds(base + i * CHUNK, CHUNK)],
                            rows.at[i % 2])
            pltpu.async_copy(rows.at[i % 2],
                             out_hbm.at[c, t, pl.ds(base + i * CHUNK, CHUNK)],
                             ssem)
        if t + 1 < T:
            _zero_acc()
        for _ in range(WCH):
            _drain(ssem)
        if t + 1 < T:
            plsc.subcore_barrier()


def _row_spec():
    return pl.BlockSpec((BLK, D), lambda i: (i, 0))


def _col_spec():
    return pl.BlockSpec((BLK, 1), lambda i: (i, 0))


def _full_spec(shape):
    return pl.BlockSpec(shape, lambda i: tuple(0 for _ in shape))


def _tc_scale_mm_body(x_ref, w_ref, p0_ref, p1_ref, o_ref):
    dinv = lax.rsqrt(p0_ref[...] + p1_ref[...] + 1.0)
    h = jnp.dot(x_ref[...], w_ref[...], preferred_element_type=jnp.float32)
    o_ref[...] = h * dinv


def _tc_scale_mm(x, w, p0, p1):
    return pl.pallas_call(
        _tc_scale_mm_body,
        grid=(GRID,),
        in_specs=[_row_spec(), _full_spec((D, D)), _col_spec(), _col_spec()],
        out_specs=_row_spec(),
        out_shape=jax.ShapeDtypeStruct((TNP, D), jnp.float32),
    )(x, w, p0, p1)


def _tc_mid_body(q0_ref, q1_ref, hs_ref, p0_ref, p1_ref, b_ref, w_ref, o_ref):
    dinv = lax.rsqrt(p0_ref[...] + p1_ref[...] + 1.0)
    o1 = dinv * (q0_ref[...] + q1_ref[...] + hs_ref[...]) + b_ref[...]
    o1 = jnp.maximum(o1, 0.0)
    h = jnp.dot(o1, w_ref[...], preferred_element_type=jnp.float32)
    o_ref[...] = h * dinv


def _tc_mid(q0, q1, hs, p0, p1, b, w):
    return pl.pallas_call(
        _tc_mid_body,
        grid=(GRID,),
        in_specs=[_row_spec(), _row_spec(), _row_spec(), _col_spec(),
                  _col_spec(), _full_spec((1, D)), _full_spec((D, D))],
        out_specs=_row_spec(),
        out_shape=jax.ShapeDtypeStruct((TNP, D), jnp.float32),
    )(q0, q1, hs, p0, p1, b, w)


def _tc_final_body(q0_ref, q1_ref, hs_ref, p0_ref, p1_ref, b_ref, wc_ref,
                   bc_ref, o_ref, imp_ref):
    dinv = lax.rsqrt(p0_ref[...] + p1_ref[...] + 1.0)
    o2 = dinv * (q0_ref[...] + q1_ref[...] + hs_ref[...]) + b_ref[...]
    o_ref[...] = o2
    imp_ref[...] = jnp.dot(o2, wc_ref[...],
                           preferred_element_type=jnp.float32) + bc_ref[...]


def _tc_final(q0, q1, hs, p0, p1, b, wc, bc):
    return pl.pallas_call(
        _tc_final_body,
        grid=(GRID,),
        in_specs=[_row_spec(), _row_spec(), _row_spec(), _col_spec(),
                  _col_spec(), _full_spec((1, D)), _full_spec((D, 1)),
                  _full_spec((1, 1))],
        out_specs=[_row_spec(), _col_spec()],
        out_shape=[jax.ShapeDtypeStruct((TNP, D), jnp.float32),
                   jax.ShapeDtypeStruct((TNP, 1), jnp.float32)],
    )(q0, q1, hs, p0, p1, b, wc, bc)


def kernel(x_seq, edge_indices, W1, b1, W2, b2, Wc, bc):
    x2 = jnp.pad(x_seq, ((0, 0), (0, NP - N), (0, 0))).reshape(TNP, D)
    pad_idx = N + (jnp.arange(E_PAD - E, dtype=jnp.int32) % (NP - N))

    def _tiled(a):
        return jnp.concatenate([a, pad_idx]).reshape(NW, NCH, CHUNK)

    src_t = [_tiled(edge_indices[t, 0]) for t in range(T)]
    dst_t = [_tiled(edge_indices[t, 1]) for t in range(T)]
    deg_idx = jnp.concatenate([dst_t[0], dst_t[1] + NP], axis=1)
    src_all = jnp.stack([src_t[0], src_t[1] + NP])
    dst_all = jnp.stack([dst_t[0], dst_t[1]])

    degp = _deg_kernel(deg_idx)
    p0 = degp[0][:, None]
    p1 = degp[1][:, None]
    b1r = b1.reshape(1, D)
    b2r = b2.reshape(1, D)
    bcr = bc.reshape(1, 1)

    hs1 = _tc_scale_mm(x2, W1, p0, p1)
    q = _agg_kernel(hs1, src_all, dst_all)
    hs2 = _tc_mid(q[0].reshape(TNP, D), q[1].reshape(TNP, D), hs1, p0, p1,
                  b1r, W2)
    q2 = _agg_kernel(hs2, src_all, dst_all)
    o2, imp = _tc_final(q2[0].reshape(TNP, D), q2[1].reshape(TNP, D), hs2,
                        p0, p1, b2r, Wc, bcr)
    return imp[NP:NP + N, 0], jnp.stack([o2[0:N], o2[NP:NP + N]])


# speedup vs baseline: 1.1161x; 1.0617x over previous
import functools

import jax
import jax.numpy as jnp
from jax import lax
from jax.experimental import pallas as pl
from jax.experimental.pallas import tpu as pltpu
from jax.experimental.pallas import tpu_sc as plsc

T = 2
N = 10000
E = 320000
D = 128

NC = 2
NS = 16
NW = NC * NS

NP = 10240
TNP = T * NP
CHUNK = 128
EPT = 10112
NCH = EPT // CHUNK
E_PAD = NW * EPT

ZR = 32
IDX_BLK = 40
PHASES = ((0, 40), (40, 39))
ROWS_PER_TILE = NP // NS
WCH = ROWS_PER_TILE // CHUNK
DEG_SLICE = TNP // NS
DEG_WIN = 8

BLK = 1280
GRID = TNP // BLK

_mesh = plsc.VectorSubcoreMesh(
    core_axis_name="c", subcore_axis_name="s", num_cores=NC, num_subcores=NS)


@functools.partial(
    pl.kernel,
    out_type=jax.ShapeDtypeStruct((NC, TNP), jnp.float32),
    mesh=_mesh,
    scratch_types=[
        pltpu.VMEM((T * NCH, CHUNK), jnp.int32),
        pltpu.VMEM((CHUNK,), jnp.float32),
        pltpu.VMEM((DEG_SLICE,), jnp.float32),
        pltpu.VMEM_SHARED((TNP,), jnp.float32),
        pltpu.SemaphoreType.DMA,
    ],
)
def _deg_kernel(dst_hbm, out_hbm, idx_v, ones_v, buf_v, deg_sh, dsem):
    c = lax.axis_index("c")
    s = lax.axis_index("s")
    wid = s * NC + c
    for i in range(CHUNK // 16):
        ones_v[pl.ds(i * 16, 16)] = jnp.ones((16,), jnp.float32)
    for i in range(DEG_SLICE // 16):
        buf_v[pl.ds(i * 16, 16)] = jnp.zeros((16,), jnp.float32)
    pltpu.sync_copy(buf_v, deg_sh.at[pl.ds(s * DEG_SLICE, DEG_SLICE)])
    pltpu.sync_copy(dst_hbm.at[wid], idx_v)
    plsc.subcore_barrier()

    def _drain():
        pltpu.make_async_copy(out_hbm.at[0, pl.ds(0, CHUNK)], ones_v,
                              dsem).wait()

    @pl.loop(0, T * NCH)
    def _scatter(j):
        pltpu.async_copy(ones_v, deg_sh.at[idx_v.at[j]], dsem, add=True)

        @pl.when(j >= DEG_WIN)
        def _():
            _drain()

    for _ in range(DEG_WIN):
        _drain()
    plsc.subcore_barrier()
    pltpu.sync_copy(deg_sh.at[pl.ds(s * DEG_SLICE, DEG_SLICE)], buf_v)
    pltpu.sync_copy(buf_v, out_hbm.at[c, pl.ds(s * DEG_SLICE, DEG_SLICE)])


@functools.partial(
    pl.kernel,
    out_type=jax.ShapeDtypeStruct((NC, NP, D), jnp.float32),
    mesh=_mesh,
    scratch_types=[
        pltpu.VMEM((IDX_BLK, CHUNK), jnp.int32),
        pltpu.VMEM((IDX_BLK, CHUNK), jnp.int32),
        pltpu.VMEM((2, CHUNK, D), jnp.float32),
        pltpu.VMEM((ZR, D), jnp.float32),
        pltpu.VMEM_SHARED((NP, D), jnp.float32),
        pltpu.SemaphoreType.DMA,
        pltpu.SemaphoreType.DMA,
    ],
)
def _agg_kernel(hs_hbm, src_hbm, dst_hbm, out_hbm, srcv, dstv, rows, zbuf,
                acc, gsem, ssem):
    c = lax.axis_index("c")
    s = lax.axis_index("s")
    wid = s * NC + c
    base = s * ROWS_PER_TILE

    for r in range(ZR):
        for k in range(D // 16):
            zbuf[r, pl.ds(k * 16, 16)] = jnp.zeros((16,), jnp.float32)

    def _drain(sem):
        pltpu.make_async_copy(hs_hbm.at[pl.ds(0, CHUNK)], rows.at[0],
                              sem).wait()

    for i in range(ROWS_PER_TILE // ZR):
        pltpu.sync_copy(zbuf, acc.at[pl.ds(base + i * ZR, ZR)])
    plsc.subcore_barrier()

    for off, cnt in PHASES:
        pltpu.sync_copy(src_hbm.at[wid, pl.ds(off, cnt)],
                        srcv.at[pl.ds(0, cnt)])
        pltpu.sync_copy(dst_hbm.at[wid, pl.ds(off, cnt)],
                        dstv.at[pl.ds(0, cnt)])
        pltpu.async_copy(hs_hbm.at[srcv.at[0]], rows.at[0], gsem)
        pltpu.async_copy(hs_hbm.at[srcv.at[1]], rows.at[1], gsem)

        @pl.loop(0, cnt)
        def _edges(j):
            _drain(gsem)
            pltpu.async_copy(rows.at[j % 2], acc.at[dstv.at[j]], ssem,
                             add=True)
            @pl.when(j + 2 < cnt)
            def _():
                _drain(ssem)
                pltpu.async_copy(hs_hbm.at[srcv.at[j + 2]],
                                 rows.at[j % 2], gsem)

        _drain(ssem)
        _drain(ssem)

    plsc.subcore_barrier()
    for i in range(WCH):
        pltpu.sync_copy(acc.at[pl.ds(base + i * CHUNK, CHUNK)],
                        rows.at[i % 2])
        pltpu.async_copy(rows.at[i % 2],
                         out_hbm.at[c, pl.ds(base + i * CHUNK, CHUNK)],
                         ssem)
    for _ in range(WCH):
        _drain(ssem)


def _row_spec(off=0):
    return pl.BlockSpec((BLK, D), lambda i, off=off: (i + off, 0))


def _col_spec(off=0):
    return pl.BlockSpec((BLK, 1), lambda i, off=off: (i + off, 0))


def _full_spec(shape):
    return pl.BlockSpec(shape, lambda i: tuple(0 for _ in shape))


def _tc_scale_mm_body(x_ref, w_ref, p0_ref, p1_ref, o_ref):
    dinv = lax.rsqrt(p0_ref[...] + p1_ref[...] + 1.0)
    h = jnp.dot(x_ref[...], w_ref[...], preferred_element_type=jnp.float32)
    o_ref[...] = h * dinv


def _tc_scale_mm(x, w, p0, p1):
    return pl.pallas_call(
        _tc_scale_mm_body,
        grid=(GRID,),
        in_specs=[_row_spec(), _full_spec((D, D)), _col_spec(), _col_spec()],
        out_specs=_row_spec(),
        out_shape=jax.ShapeDtypeStruct((TNP, D), jnp.float32),
    )(x, w, p0, p1)


def _tc_mid_body(q0_ref, q1_ref, hs_ref, p0_ref, p1_ref, b_ref, w_ref, o_ref):
    dinv = lax.rsqrt(p0_ref[...] + p1_ref[...] + 1.0)
    o1 = dinv * (q0_ref[...] + q1_ref[...] + hs_ref[...]) + b_ref[...]
    o1 = jnp.maximum(o1, 0.0)
    h = jnp.dot(o1, w_ref[...], preferred_element_type=jnp.float32)
    o_ref[...] = h * dinv


def _tc_mid(q0, q1, hs, p0, p1, b, w, toff):
    return pl.pallas_call(
        _tc_mid_body,
        grid=(NP // BLK,),
        in_specs=[_row_spec(), _row_spec(), _row_spec(toff), _col_spec(toff),
                  _col_spec(toff), _full_spec((1, D)), _full_spec((D, D))],
        out_specs=_row_spec(),
        out_shape=jax.ShapeDtypeStruct((NP, D), jnp.float32),
    )(q0, q1, hs, p0, p1, b, w)


def _tc_final_body(q0_ref, q1_ref, hs_ref, p0_ref, p1_ref, b_ref, wc_ref,
                   bc_ref, o_ref, imp_ref):
    dinv = lax.rsqrt(p0_ref[...] + p1_ref[...] + 1.0)
    o2 = dinv * (q0_ref[...] + q1_ref[...] + hs_ref[...]) + b_ref[...]
    o_ref[...] = o2
    imp_ref[...] = jnp.dot(o2, wc_ref[...],
                           preferred_element_type=jnp.float32) + bc_ref[...]


def _tc_final(q0, q1, hs, p0, p1, b, wc, bc, toff):
    return pl.pallas_call(
        _tc_final_body,
        grid=(NP // BLK,),
        in_specs=[_row_spec(), _row_spec(), _row_spec(), _col_spec(toff),
                  _col_spec(toff), _full_spec((1, D)), _full_spec((D, 1)),
                  _full_spec((1, 1))],
        out_specs=[_row_spec(), _col_spec()],
        out_shape=[jax.ShapeDtypeStruct((NP, D), jnp.float32),
                   jax.ShapeDtypeStruct((NP, 1), jnp.float32)],
    )(q0, q1, hs, p0, p1, b, wc, bc)


def kernel(x_seq, edge_indices, W1, b1, W2, b2, Wc, bc):
    x2 = jnp.pad(x_seq, ((0, 0), (0, NP - N), (0, 0))).reshape(TNP, D)
    pad_idx = N + (jnp.arange(E_PAD - E, dtype=jnp.int32) % (NP - N))

    def _tiled(a):
        return jnp.concatenate([a, pad_idx]).reshape(NW, NCH, CHUNK)

    src_t = [_tiled(edge_indices[t, 0]) for t in range(T)]
    dst_t = [_tiled(edge_indices[t, 1]) for t in range(T)]
    deg_idx = jnp.concatenate([dst_t[0], dst_t[1] + NP], axis=1)
    src_off = [src_t[0], src_t[1] + NP]

    degp = _deg_kernel(deg_idx)
    p0 = degp[0][:, None]
    p1 = degp[1][:, None]
    b1r = b1.reshape(1, D)
    b2r = b2.reshape(1, D)
    bcr = bc.reshape(1, 1)
    tb = NP // BLK

    hs1 = _tc_scale_mm(x2, W1, p0, p1)
    q1t = [_agg_kernel(hs1, src_off[t], dst_t[t]) for t in range(T)]
    hs2 = [_tc_mid(q1t[t][0], q1t[t][1], hs1, p0, p1, b1r, W2, t * tb)
           for t in range(T)]
    q2t = [_agg_kernel(hs2[t], src_t[t], dst_t[t]) for t in range(T)]
    fin = [_tc_final(q2t[t][0], q2t[t][1], hs2[t], p0, p1, b2r, Wc, bcr,
                     t * tb) for t in range(T)]
    return fin[1][1][:N, 0], jnp.stack([fin[0][0][:N], fin[1][0][:N]])


# speedup vs baseline: 1.1546x; 1.0344x over previous
import functools

import jax
import jax.numpy as jnp
from jax import lax
from jax.experimental import pallas as pl
from jax.experimental.pallas import tpu as pltpu
from jax.experimental.pallas import tpu_sc as plsc

T = 2
N = 10000
E = 320000
D = 128

NC = 2
NS = 16
NW = NC * NS

NP = 10240
TNP = T * NP
EPT = 10240
E_PAD = NW * EPT

DCHUNK = 128
DNCH = EPT // DCHUNK

ACH = 64
NPH = 4
APH = EPT // ACH // NPH
NBUF = 4

ZR = 8
ROWS_PER_TILE = NP // NS
WCH = ROWS_PER_TILE // ACH
DEG_SLICE = TNP // NS
DEG_WIN = 8

BLK = 1280
GRID = TNP // BLK

_mesh = plsc.VectorSubcoreMesh(
    core_axis_name="c", subcore_axis_name="s", num_cores=NC, num_subcores=NS)


@functools.partial(
    pl.kernel,
    out_type=jax.ShapeDtypeStruct((NC, TNP), jnp.float32),
    mesh=_mesh,
    scratch_types=[
        pltpu.VMEM((T * DNCH, DCHUNK), jnp.int32),
        pltpu.VMEM((DCHUNK,), jnp.float32),
        pltpu.VMEM((DEG_SLICE,), jnp.float32),
        pltpu.VMEM_SHARED((TNP,), jnp.float32),
        pltpu.SemaphoreType.DMA,
    ],
)
def _deg_kernel(dst_hbm, out_hbm, idx_v, ones_v, buf_v, deg_sh, dsem):
    c = lax.axis_index("c")
    s = lax.axis_index("s")
    wid = s * NC + c
    for i in range(DCHUNK // 16):
        ones_v[pl.ds(i * 16, 16)] = jnp.ones((16,), jnp.float32)
    for i in range(DEG_SLICE // 16):
        buf_v[pl.ds(i * 16, 16)] = jnp.zeros((16,), jnp.float32)
    pltpu.sync_copy(buf_v, deg_sh.at[pl.ds(s * DEG_SLICE, DEG_SLICE)])
    pltpu.sync_copy(dst_hbm.at[wid], idx_v)
    plsc.subcore_barrier()

    def _drain():
        pltpu.make_async_copy(out_hbm.at[0, pl.ds(0, DCHUNK)], ones_v,
                              dsem).wait()

    @pl.loop(0, T * DNCH)
    def _scatter(j):
        pltpu.async_copy(ones_v, deg_sh.at[idx_v.at[j]], dsem, add=True)

        @pl.when(j >= DEG_WIN)
        def _():
            _drain()

    for _ in range(DEG_WIN):
        _drain()
    plsc.subcore_barrier()
    pltpu.sync_copy(deg_sh.at[pl.ds(s * DEG_SLICE, DEG_SLICE)], buf_v)
    pltpu.sync_copy(buf_v, out_hbm.at[c, pl.ds(s * DEG_SLICE, DEG_SLICE)])


@functools.partial(
    pl.kernel,
    out_type=jax.ShapeDtypeStruct((NC, NP, D), jnp.float32),
    mesh=_mesh,
    scratch_types=[
        pltpu.VMEM((APH, ACH), jnp.int32),
        pltpu.VMEM((APH, ACH), jnp.int32),
        pltpu.VMEM((NBUF, ACH, D), jnp.float32),
        pltpu.VMEM((ZR, D), jnp.float32),
        pltpu.VMEM_SHARED((NP, D), jnp.float32),
        pltpu.SemaphoreType.DMA,
        pltpu.SemaphoreType.DMA,
    ],
)
def _agg_kernel(hs_hbm, src_hbm, dst_hbm, out_hbm, srcv, dstv, rows, zbuf,
                acc, gsem, ssem):
    c = lax.axis_index("c")
    s = lax.axis_index("s")
    wid = s * NC + c
    base = s * ROWS_PER_TILE

    for r in range(ZR):
        for k in range(D // 16):
            zbuf[r, pl.ds(k * 16, 16)] = jnp.zeros((16,), jnp.float32)

    def _drain(sem):
        pltpu.make_async_copy(hs_hbm.at[pl.ds(0, ACH)], rows.at[0],
                              sem).wait()

    for i in range(ROWS_PER_TILE // ZR):
        pltpu.sync_copy(zbuf, acc.at[pl.ds(base + i * ZR, ZR)])
    plsc.subcore_barrier()

    for ph in range(NPH):
        pltpu.sync_copy(src_hbm.at[wid, ph], srcv)
        pltpu.sync_copy(dst_hbm.at[wid, ph], dstv)
        pltpu.async_copy(hs_hbm.at[srcv.at[0]], rows.at[0], gsem)
        pltpu.async_copy(hs_hbm.at[srcv.at[1]], rows.at[1], gsem)
        pltpu.async_copy(hs_hbm.at[srcv.at[2]], rows.at[2], gsem)

        @pl.loop(0, APH)
        def _edges(j):
            _drain(gsem)
            pltpu.async_copy(rows.at[j % NBUF], acc.at[dstv.at[j]], ssem,
                             add=True)
            @pl.when(j + 3 < APH)
            def _():
                @pl.when(j >= 1)
                def _():
                    _drain(ssem)

                pltpu.async_copy(hs_hbm.at[srcv.at[j + 3]],
                                 rows.at[(j + 3) % NBUF], gsem)

        for _ in range(NBUF):
            _drain(ssem)

    plsc.subcore_barrier()
    for i in range(WCH):
        if i >= NBUF:
            _drain(ssem)
        pltpu.sync_copy(acc.at[pl.ds(base + i * ACH, ACH)],
                        rows.at[i % NBUF])
        pltpu.async_copy(rows.at[i % NBUF],
                         out_hbm.at[c, pl.ds(base + i * ACH, ACH)],
                         ssem)
    for _ in range(NBUF):
        _drain(ssem)


def _row_spec(off=0):
    return pl.BlockSpec((BLK, D), lambda i, off=off: (i + off, 0))


def _col_spec(off=0):
    return pl.BlockSpec((BLK, 1), lambda i, off=off: (i + off, 0))


def _full_spec(shape):
    return pl.BlockSpec(shape, lambda i: tuple(0 for _ in shape))


def _tc_scale_mm_body(x_ref, w_ref, p0_ref, p1_ref, o_ref):
    dinv = lax.rsqrt(p0_ref[...] + p1_ref[...] + 1.0)
    h = jnp.dot(x_ref[...], w_ref[...], preferred_element_type=jnp.float32)
    o_ref[...] = h * dinv


def _tc_scale_mm(x, w, p0, p1):
    return pl.pallas_call(
        _tc_scale_mm_body,
        grid=(GRID,),
        in_specs=[_row_spec(), _full_spec((D, D)), _col_spec(), _col_spec()],
        out_specs=_row_spec(),
        out_shape=jax.ShapeDtypeStruct((TNP, D), jnp.float32),
    )(x, w, p0, p1)


def _tc_mid_body(q0_ref, q1_ref, hs_ref, p0_ref, p1_ref, b_ref, w_ref, o_ref):
    dinv = lax.rsqrt(p0_ref[...] + p1_ref[...] + 1.0)
    o1 = dinv * (q0_ref[...] + q1_ref[...] + hs_ref[...]) + b_ref[...]
    o1 = jnp.maximum(o1, 0.0)
    h = jnp.dot(o1, w_ref[...], preferred_element_type=jnp.float32)
    o_ref[...] = h * dinv


def _tc_mid(q0, q1, hs, p0, p1, b, w, toff):
    return pl.pallas_call(
        _tc_mid_body,
        grid=(NP // BLK,),
        in_specs=[_row_spec(), _row_spec(), _row_spec(toff), _col_spec(toff),
                  _col_spec(toff), _full_spec((1, D)), _full_spec((D, D))],
        out_specs=_row_spec(),
        out_shape=jax.ShapeDtypeStruct((NP, D), jnp.float32),
    )(q0, q1, hs, p0, p1, b, w)


def _tc_final_body(q0_ref, q1_ref, hs_ref, p0_ref, p1_ref, b_ref, wc_ref,
                   bc_ref, o_ref, imp_ref):
    dinv = lax.rsqrt(p0_ref[...] + p1_ref[...] + 1.0)
    o2 = dinv * (q0_ref[...] + q1_ref[...] + hs_ref[...]) + b_ref[...]
    o_ref[...] = o2
    imp_ref[...] = jnp.dot(o2, wc_ref[...],
                           preferred_element_type=jnp.float32) + bc_ref[...]


def _tc_final(q0, q1, hs, p0, p1, b, wc, bc, toff):
    return pl.pallas_call(
        _tc_final_body,
        grid=(NP // BLK,),
        in_specs=[_row_spec(), _row_spec(), _row_spec(), _col_spec(toff),
                  _col_spec(toff), _full_spec((1, D)), _full_spec((D, 1)),
                  _full_spec((1, 1))],
        out_specs=[_row_spec(), _col_spec()],
        out_shape=[jax.ShapeDtypeStruct((NP, D), jnp.float32),
                   jax.ShapeDtypeStruct((NP, 1), jnp.float32)],
    )(q0, q1, hs, p0, p1, b, wc, bc)


def kernel(x_seq, edge_indices, W1, b1, W2, b2, Wc, bc):
    x2 = jnp.pad(x_seq, ((0, 0), (0, NP - N), (0, 0))).reshape(TNP, D)
    pad_idx = N + (jnp.arange(E_PAD - E, dtype=jnp.int32) % (NP - N))

    def _tiled(a, *shape):
        return jnp.concatenate([a, pad_idx]).reshape(NW, *shape)

    src_t = [_tiled(edge_indices[t, 0], NPH, APH, ACH) for t in range(T)]
    dst_t = [_tiled(edge_indices[t, 1], NPH, APH, ACH) for t in range(T)]
    dst_d = [_tiled(edge_indices[t, 1], DNCH, DCHUNK) for t in range(T)]
    deg_idx = jnp.concatenate([dst_d[0], dst_d[1] + NP], axis=1)
    src_off = [src_t[0], src_t[1] + NP]

    degp = _deg_kernel(deg_idx)
    p0 = degp[0][:, None]
    p1 = degp[1][:, None]
    b1r = b1.reshape(1, D)
    b2r = b2.reshape(1, D)
    bcr = bc.reshape(1, 1)
    tb = NP // BLK

    hs1 = _tc_scale_mm(x2, W1, p0, p1)
    q1t = [_agg_kernel(hs1, src_off[t], dst_t[t]) for t in range(T)]
    hs2 = [_tc_mid(q1t[t][0], q1t[t][1], hs1, p0, p1, b1r, W2, t * tb)
           for t in range(T)]
    q2t = [_agg_kernel(hs2[t], src_t[t], dst_t[t]) for t in range(T)]
    fin = [_tc_final(q2t[t][0], q2t[t][1], hs2[t], p0, p1, b2r, Wc, bcr,
                     t * tb) for t in range(T)]
    return fin[1][1][:N, 0], jnp.stack([fin[0][0][:N], fin[1][0][:N]])
